# baseline (speedup 1.0000x reference)
import functools

import jax
import jax.numpy as jnp
from jax import lax
from jax.experimental import pallas as pl
from jax.experimental.pallas import tpu as pltpu, tpu_sc as plsc

NC = 2
NS = 16
NW = NC * NS
GRP = 128


def _ceil_to(x, m):
    return (x + m - 1) // m * m


def _make_sc_agg(n_nodes_pad, g_slow, g_fast):
    rows_per_tile = n_nodes_pad // NS
    assert g_slow % 8 == 0 and g_fast % 8 == 0
    g_max = max(g_slow, g_fast)

    mesh = plsc.VectorSubcoreMesh(
        core_axis_name="c", subcore_axis_name="s", num_cores=NC, num_subcores=NS
    )
    out_type = [jax.ShapeDtypeStruct((n_nodes_pad, 128), jnp.float32),
                jax.ShapeDtypeStruct((n_nodes_pad, 128), jnp.float32)]
    scratch = [
        pltpu.VMEM((g_max, GRP), jnp.int32),
        pltpu.VMEM((g_max, GRP), jnp.int32),
        pltpu.VMEM((GRP, 128), jnp.float32),
        pltpu.VMEM_SHARED((n_nodes_pad, 128), jnp.float32),
        pltpu.SemaphoreType.DMA,
    ]

    def body(table, src2d, dst2d, zrows_hbm, agg_out0, agg_out1,
             srci, dsti, rows, acc, sem):
        c = lax.axis_index("c")
        s = lax.axis_index("s")
        g0 = jnp.where(c == 0, s * g_slow, NS * g_slow + s * g_fast)
        ng = jnp.where(c == 0, g_slow, g_fast)

        base_r = s * rows_per_tile
        pltpu.sync_copy(zrows_hbm, acc.at[pl.ds(base_r, rows_per_tile)])
        pltpu.sync_copy(src2d.at[pl.ds(g0, g_max)], srci)
        pltpu.sync_copy(dst2d.at[pl.ds(g0, g_max)], dsti)
        plsc.subcore_barrier()

        def step(j, carry):
            pltpu.async_copy(table.at[srci.at[j]], rows, sem).wait()
            pltpu.sync_copy(rows, acc.at[dsti.at[j]], add=True)
            return carry

        lax.fori_loop(0, ng, step, 0)
        plsc.subcore_barrier()

        @pl.when(c == 0)
        def _():
            pltpu.sync_copy(acc.at[pl.ds(base_r, rows_per_tile)],
                            agg_out0.at[pl.ds(base_r, rows_per_tile)])

        @pl.when(c == 1)
        def _():
            pltpu.sync_copy(acc.at[pl.ds(base_r, rows_per_tile)],
                            agg_out1.at[pl.ds(base_r, rows_per_tile)])

    return pl.kernel(body, out_type=out_type, mesh=mesh, scratch_types=scratch)


def _make_sc_pair_gather(n_pairs_pad, n_groups):
    mesh = plsc.VectorSubcoreMesh(
        core_axis_name="c", subcore_axis_name="s", num_cores=NC, num_subcores=NS
    )
    out_type = [
        jax.ShapeDtypeStruct((n_pairs_pad, 128), jnp.float32),
        jax.ShapeDtypeStruct((n_pairs_pad, 128), jnp.float32),
    ]
    scratch = [
        pltpu.VMEM((n_groups, GRP), jnp.int32),
        pltpu.VMEM((n_groups, GRP), jnp.int32),
        pltpu.VMEM((GRP, 128), jnp.float32),
        pltpu.VMEM((GRP, 128), jnp.float32),
        pltpu.SemaphoreType.DMA,
        pltpu.SemaphoreType.DMA,
    ]
    per_worker = n_groups * GRP

    def body(u, v, p0, p1, gu, gv, p0i, p1i, rows0a, rows0b, sem0, sem1):
        c = lax.axis_index("c")
        s = lax.axis_index("s")
        wid = c * NS + s
        pltpu.sync_copy(p0.at[wid], p0i)
        pltpu.sync_copy(p1.at[wid], p1i)

        def step(j, carry):
            out_r = wid * per_worker + j * GRP
            cp0 = pltpu.async_copy(u.at[p0i.at[j]], rows0a, sem0)
            cp1 = pltpu.async_copy(v.at[p1i.at[j]], rows0b, sem1)
            cp0.wait()
            pltpu.sync_copy(rows0a, gu.at[pl.ds(out_r, GRP)])
            cp1.wait()
            pltpu.sync_copy(rows0b, gv.at[pl.ds(out_r, GRP)])
            return carry

        lax.fori_loop(0, n_groups, step, 0)

    return pl.kernel(body, out_type=out_type, mesh=mesh, scratch_types=scratch)


def _deg_body(dst_ref, H):
    i = pl.program_id(0)

    @pl.when(i == 0)
    def _():
        H[...] = jnp.zeros_like(H)

    d = dst_ref[...]
    ids = lax.broadcasted_iota(jnp.int32, (d.shape[0], 128), 1)
    oh_hi = ((d // 128) == ids).astype(jnp.bfloat16)
    oh_lo = ((d % 128) == ids).astype(jnp.bfloat16)
    H[...] += lax.dot_general(oh_hi, oh_lo, (((0,), (0,)), ((), ())),
                              preferred_element_type=jnp.float32)


def _tc_degree(dst_col, eb=8000):
    e = dst_col.shape[0]
    return pl.pallas_call(
        _deg_body,
        grid=(e // eb,),
        in_specs=[pl.BlockSpec((eb, 1), lambda i: (i, 0))],
        out_specs=pl.BlockSpec((128, 128), lambda i: (0, 0)),
        out_shape=jax.ShapeDtypeStruct((128, 128), jnp.float32),
    )(dst_col)


def _layer1_body(a0, a1, dcol, x, wl, wr, b, o):
    deg = jnp.maximum(dcol[...], 1.0)
    mean = (a0[...] + a1[...]) / deg
    acc = jnp.dot(mean, wl[...], preferred_element_type=jnp.float32)
    acc += jnp.dot(x[...], wr[...], preferred_element_type=jnp.float32)
    o[...] = jnp.maximum(acc + b[...], 0.0)


def _layer2_body(a0, a1, dcol, x, wl, wr, b, wsa, wsb, u, v):
    deg = jnp.maximum(dcol[...], 1.0)
    mean = (a0[...] + a1[...]) / deg
    acc = jnp.dot(mean, wl[...], preferred_element_type=jnp.float32)
    acc += jnp.dot(x[...], wr[...], preferred_element_type=jnp.float32)
    h2 = jnp.maximum(acc + b[...], 0.0)
    u[...] = jnp.dot(h2, wsa[...], preferred_element_type=jnp.float32)
    v[...] = jnp.dot(h2, wsb[...], preferred_element_type=jnp.float32)


def _score_body(gu, gv, bs1, ws2, bs2, o):
    z = jnp.maximum(gu[...] + gv[...] + bs1[...], 0.0)
    s = jnp.dot(z, ws2[...], preferred_element_type=jnp.float32) + bs2[...]
    o[...] = 1.0 / (1.0 + jnp.exp(-s))


def _row_spec(rb, cols):
    return pl.BlockSpec((rb, cols), lambda i: (i, 0))


def _full_spec(shape):
    return pl.BlockSpec(shape, lambda i: (0,) * len(shape))


def _tc_layer1(a0, a1, dcol, x, wl, wr, b, rb=1000):
    n = x.shape[0]
    return pl.pallas_call(
        _layer1_body,
        grid=(n // rb,),
        in_specs=[_row_spec(rb, 128), _row_spec(rb, 128), _row_spec(rb, 1),
                  _row_spec(rb, 128), _full_spec((128, 128)),
                  _full_spec((128, 128)), _full_spec((1, 128))],
        out_specs=_row_spec(rb, 128),
        out_shape=jax.ShapeDtypeStruct((n, 128), jnp.float32),
    )(a0, a1, dcol, x, wl, wr, b)


def _tc_layer2(a0, a1, dcol, x, wl, wr, b, wsa, wsb, rb=1000):
    n = x.shape[0]
    return pl.pallas_call(
        _layer2_body,
        grid=(n // rb,),
        in_specs=[_row_spec(rb, 128), _row_spec(rb, 128), _row_spec(rb, 1),
                  _row_spec(rb, 128), _full_spec((128, 128)),
                  _full_spec((128, 128)), _full_spec((1, 128)),
                  _full_spec((128, 128)), _full_spec((128, 128))],
        out_specs=[_row_spec(rb, 128), _row_spec(rb, 128)],
        out_shape=[jax.ShapeDtypeStruct((n, 128), jnp.float32),
                   jax.ShapeDtypeStruct((n, 128), jnp.float32)],
    )(a0, a1, dcol, x, wl, wr, b, wsa, wsb)


def _tc_score(gu, gv, n, bs1, ws2, bs2, rb=1000):
    return pl.pallas_call(
        _score_body,
        grid=(n // rb,),
        in_specs=[_row_spec(rb, 128), _row_spec(rb, 128), _full_spec((1, 128)),
                  _full_spec((128, 1)), _full_spec((1, 1))],
        out_specs=_row_spec(rb, 1),
        out_shape=jax.ShapeDtypeStruct((n, 1), jnp.float32),
    )(gu, gv, bs1, ws2, bs2)


def kernel(x, edge_index, pairs, Wl1, Wr1, b1, Wl2, Wr2, b2, Ws1, bs1, Ws2, bs2):
    n_nodes = x.shape[0]
    n_edges = edge_index.shape[1]
    n_pairs = pairs.shape[0]

    g_slow, g_fast = 56, 104
    e_groups_total = NS * (g_slow + g_fast)
    e_pad = e_groups_total * GRP
    assert e_pad >= n_edges
    node_pad = _ceil_to(n_nodes + 1, NS * 8)
    p_pad = _ceil_to(n_pairs, NW * GRP)
    p_groups = p_pad // (NW * GRP)

    src = jnp.pad(edge_index[0].astype(jnp.int32), (0, e_pad - n_edges),
                  constant_values=0).reshape(e_groups_total, GRP)
    dst = jnp.pad(edge_index[1].astype(jnp.int32), (0, e_pad - n_edges),
                  constant_values=n_nodes).reshape(e_groups_total, GRP)
    p0 = jnp.pad(pairs[:, 0].astype(jnp.int32), (0, p_pad - n_pairs),
                 constant_values=0).reshape(NW, p_groups, GRP)
    p1 = jnp.pad(pairs[:, 1].astype(jnp.int32), (0, p_pad - n_pairs),
                 constant_values=0).reshape(NW, p_groups, GRP)

    rows_per_tile = node_pad // NS
    zrows_hbm = jnp.zeros((rows_per_tile, 128), jnp.float32)

    sc_agg = _make_sc_agg(node_pad, g_slow, g_fast)
    sc_pairs = _make_sc_pair_gather(p_pad, p_groups)

    a0, a1 = sc_agg(x, src, dst, zrows_hbm)
    H = _tc_degree(edge_index[1].astype(jnp.int32).reshape(n_edges, 1))
    dcol = H.reshape(-1)[:n_nodes, None]

    b1r = b1.reshape(1, 128)
    h = _tc_layer1(a0, a1, dcol, x, Wl1, Wr1, b1r)

    a20, a21 = sc_agg(h, src, dst, zrows_hbm)
    u, v = _tc_layer2(a20, a21, dcol, h, Wl2, Wr2, b2.reshape(1, 128),
                      Ws1[:128], Ws1[128:])

    gu, gv = sc_pairs(u, v, p0, p1)
    s = _tc_score(gu, gv, n_pairs, bs1.reshape(1, 128), Ws2, bs2.reshape(1, 1))
    return s.squeeze(-1)


# speedup vs baseline: 1.0854x; 1.0854x over previous
import functools

import jax
import jax.numpy as jnp
from jax import lax
from jax.experimental import pallas as pl
from jax.experimental.pallas import tpu as pltpu, tpu_sc as plsc

NC = 2
NS = 16
NW = NC * NS
GRP = 128


def _ceil_to(x, m):
    return (x + m - 1) // m * m


def _make_sc_agg(n_nodes_pad, g_slow, g_fast):
    rows_per_tile = n_nodes_pad // NS
    assert g_slow % 8 == 0 and g_fast % 8 == 0
    g_max = max(g_slow, g_fast)

    mesh = plsc.VectorSubcoreMesh(
        core_axis_name="c", subcore_axis_name="s", num_cores=NC, num_subcores=NS
    )
    out_type = [jax.ShapeDtypeStruct((n_nodes_pad, 128), jnp.float32),
                jax.ShapeDtypeStruct((n_nodes_pad, 128), jnp.float32)]
    scratch = [
        pltpu.VMEM((g_max, GRP), jnp.int32),
        pltpu.VMEM((g_max, GRP), jnp.int32),
        pltpu.VMEM((GRP, 128), jnp.float32),
        pltpu.VMEM_SHARED((n_nodes_pad, 128), jnp.float32),
        pltpu.SemaphoreType.DMA,
    ]

    def body(table, src2d, dst2d, zrows_hbm, agg_out0, agg_out1,
             srci, dsti, rows, acc, sem):
        c = lax.axis_index("c")
        s = lax.axis_index("s")
        g0 = jnp.where(c == 0, s * g_slow, NS * g_slow + s * g_fast)
        ng = jnp.where(c == 0, g_slow, g_fast)

        base_r = s * rows_per_tile
        pltpu.sync_copy(zrows_hbm, acc.at[pl.ds(base_r, rows_per_tile)])
        pltpu.sync_copy(src2d.at[pl.ds(g0, g_max)], srci)
        pltpu.sync_copy(dst2d.at[pl.ds(g0, g_max)], dsti)
        plsc.subcore_barrier()

        def step(j, carry):
            pltpu.async_copy(table.at[srci.at[j]], rows, sem).wait()
            pltpu.sync_copy(rows, acc.at[dsti.at[j]], add=True)
            return carry

        lax.fori_loop(0, ng, step, 0)
        plsc.subcore_barrier()

        @pl.when(c == 0)
        def _():
            pltpu.sync_copy(acc.at[pl.ds(base_r, rows_per_tile)],
                            agg_out0.at[pl.ds(base_r, rows_per_tile)])

        @pl.when(c == 1)
        def _():
            pltpu.sync_copy(acc.at[pl.ds(base_r, rows_per_tile)],
                            agg_out1.at[pl.ds(base_r, rows_per_tile)])

    return pl.kernel(body, out_type=out_type, mesh=mesh, scratch_types=scratch)


def _make_sc_pair_gather(n_pairs_pad, n_groups):
    mesh = plsc.VectorSubcoreMesh(
        core_axis_name="c", subcore_axis_name="s", num_cores=NC, num_subcores=NS
    )
    out_type = [
        jax.ShapeDtypeStruct((n_pairs_pad, 128), jnp.float32),
        jax.ShapeDtypeStruct((n_pairs_pad, 128), jnp.float32),
    ]
    scratch = [
        pltpu.VMEM((n_groups, GRP), jnp.int32),
        pltpu.VMEM((n_groups, GRP), jnp.int32),
        pltpu.VMEM((GRP, 128), jnp.float32),
        pltpu.VMEM((GRP, 128), jnp.float32),
        pltpu.SemaphoreType.DMA,
        pltpu.SemaphoreType.DMA,
    ]
    per_worker = n_groups * GRP

    def body(u, v, p0, p1, gu, gv, p0i, p1i, rows0a, rows0b, sem0, sem1):
        c = lax.axis_index("c")
        s = lax.axis_index("s")
        wid = c * NS + s
        pltpu.sync_copy(p0.at[wid], p0i)
        pltpu.sync_copy(p1.at[wid], p1i)

        def step(j, carry):
            out_r = wid * per_worker + j * GRP
            cp0 = pltpu.async_copy(u.at[p0i.at[j]], rows0a, sem0)
            cp1 = pltpu.async_copy(v.at[p1i.at[j]], rows0b, sem1)
            cp0.wait()
            pltpu.sync_copy(rows0a, gu.at[pl.ds(out_r, GRP)])
            cp1.wait()
            pltpu.sync_copy(rows0b, gv.at[pl.ds(out_r, GRP)])
            return carry

        lax.fori_loop(0, n_groups, step, 0)

    return pl.kernel(body, out_type=out_type, mesh=mesh, scratch_types=scratch)


def _deg_body(dst_ref, H):
    i = pl.program_id(0)

    @pl.when(i == 0)
    def _():
        H[...] = jnp.zeros_like(H)

    d = dst_ref[...]
    ids = lax.broadcasted_iota(jnp.int32, (d.shape[0], 128), 1)
    oh_hi = ((d // 128) == ids).astype(jnp.bfloat16)
    oh_lo = ((d % 128) == ids).astype(jnp.bfloat16)
    H[...] += lax.dot_general(oh_hi, oh_lo, (((0,), (0,)), ((), ())),
                              preferred_element_type=jnp.float32)


def _tc_degree(dst_col, eb=8000):
    e = dst_col.shape[0]
    return pl.pallas_call(
        _deg_body,
        grid=(e // eb,),
        in_specs=[pl.BlockSpec((eb, 1), lambda i: (i, 0))],
        out_specs=pl.BlockSpec((128, 128), lambda i: (0, 0)),
        out_shape=jax.ShapeDtypeStruct((128, 128), jnp.float32),
    )(dst_col)


def _layer1_body(a0, a1, dcol, x, wl, wr, b, o):
    deg = jnp.maximum(dcol[...], 1.0)
    mean = (a0[...] + a1[...]) / deg
    acc = jnp.dot(mean, wl[...], preferred_element_type=jnp.float32)
    acc += jnp.dot(x[...], wr[...], preferred_element_type=jnp.float32)
    o[...] = jnp.maximum(acc + b[...], 0.0)


def _layer2_body(a0, a1, dcol, x, wl, wr, b, wsa, wsb, u, v):
    deg = jnp.maximum(dcol[...], 1.0)
    mean = (a0[...] + a1[...]) / deg
    acc = jnp.dot(mean, wl[...], preferred_element_type=jnp.float32)
    acc += jnp.dot(x[...], wr[...], preferred_element_type=jnp.float32)
    h2 = jnp.maximum(acc + b[...], 0.0)
    u[...] = jnp.dot(h2, wsa[...], preferred_element_type=jnp.float32)
    v[...] = jnp.dot(h2, wsb[...], preferred_element_type=jnp.float32)


def _score_body(gu, gv, bs1, ws2, bs2, o):
    z = jnp.maximum(gu[...] + gv[...] + bs1[...], 0.0)
    s = jnp.dot(z, ws2[...], preferred_element_type=jnp.float32) + bs2[...]
    o[...] = 1.0 / (1.0 + jnp.exp(-s))


def _row_spec(rb, cols):
    return pl.BlockSpec((rb, cols), lambda i: (i, 0))


def _full_spec(shape):
    return pl.BlockSpec(shape, lambda i: (0,) * len(shape))


def _tc_layer1(a0, a1, dcol, x, wl, wr, b, rb=1000):
    n = x.shape[0]
    return pl.pallas_call(
        _layer1_body,
        grid=(n // rb,),
        in_specs=[_row_spec(rb, 128), _row_spec(rb, 128), _row_spec(rb, 1),
                  _row_spec(rb, 128), _full_spec((128, 128)),
                  _full_spec((128, 128)), _full_spec((1, 128))],
        out_specs=_row_spec(rb, 128),
        out_shape=jax.ShapeDtypeStruct((n, 128), jnp.float32),
    )(a0, a1, dcol, x, wl, wr, b)


def _tc_layer2(a0, a1, dcol, x, wl, wr, b, wsa, wsb, rb=1000):
    n = x.shape[0]
    return pl.pallas_call(
        _layer2_body,
        grid=(n // rb,),
        in_specs=[_row_spec(rb, 128), _row_spec(rb, 128), _row_spec(rb, 1),
                  _row_spec(rb, 128), _full_spec((128, 128)),
                  _full_spec((128, 128)), _full_spec((1, 128)),
                  _full_spec((128, 128)), _full_spec((128, 128))],
        out_specs=[_row_spec(rb, 128), _row_spec(rb, 128)],
        out_shape=[jax.ShapeDtypeStruct((n, 128), jnp.float32),
                   jax.ShapeDtypeStruct((n, 128), jnp.float32)],
    )(a0, a1, dcol, x, wl, wr, b, wsa, wsb)


def _tc_score(gu, gv, n, bs1, ws2, bs2, rb=1000):
    return pl.pallas_call(
        _score_body,
        grid=(n // rb,),
        in_specs=[_row_spec(rb, 128), _row_spec(rb, 128), _full_spec((1, 128)),
                  _full_spec((128, 1)), _full_spec((1, 1))],
        out_specs=_row_spec(rb, 1),
        out_shape=jax.ShapeDtypeStruct((n, 1), jnp.float32),
    )(gu, gv, bs1, ws2, bs2)


def kernel(x, edge_index, pairs, Wl1, Wr1, b1, Wl2, Wr2, b2, Ws1, bs1, Ws2, bs2):
    n_nodes = x.shape[0]
    n_edges = edge_index.shape[1]
    n_pairs = pairs.shape[0]

    g_slow, g_fast = 104, 56
    e_groups_total = NS * (g_slow + g_fast)
    e_pad = e_groups_total * GRP
    assert e_pad >= n_edges
    node_pad = _ceil_to(n_nodes + 1, NS * 8)
    p_pad = _ceil_to(n_pairs, NW * GRP)
    p_groups = p_pad // (NW * GRP)

    src = jnp.pad(edge_index[0].astype(jnp.int32), (0, e_pad - n_edges),
                  constant_values=0).reshape(e_groups_total, GRP)
    dst = jnp.pad(edge_index[1].astype(jnp.int32), (0, e_pad - n_edges),
                  constant_values=n_nodes).reshape(e_groups_total, GRP)
    p0 = jnp.pad(pairs[:, 0].astype(jnp.int32), (0, p_pad - n_pairs),
                 constant_values=0).reshape(NW, p_groups, GRP)
    p1 = jnp.pad(pairs[:, 1].astype(jnp.int32), (0, p_pad - n_pairs),
                 constant_values=0).reshape(NW, p_groups, GRP)

    rows_per_tile = node_pad // NS
    zrows_hbm = jnp.zeros((rows_per_tile, 128), jnp.float32)

    sc_agg = _make_sc_agg(node_pad, g_slow, g_fast)
    sc_pairs = _make_sc_pair_gather(p_pad, p_groups)

    a0, a1 = sc_agg(x, src, dst, zrows_hbm)
    H = _tc_degree(edge_index[1].astype(jnp.int32).reshape(n_edges, 1))
    dcol = H.reshape(-1)[:n_nodes, None]

    b1r = b1.reshape(1, 128)
    h = _tc_layer1(a0, a1, dcol, x, Wl1, Wr1, b1r)

    a20, a21 = sc_agg(h, src, dst, zrows_hbm)
    u, v = _tc_layer2(a20, a21, dcol, h, Wl2, Wr2, b2.reshape(1, 128),
                      Ws1[:128], Ws1[128:])

    gu, gv = sc_pairs(u, v, p0, p1)
    s = _tc_score(gu, gv, n_pairs, bs1.reshape(1, 128), Ws2, bs2.reshape(1, 1))
    return s.squeeze(-1)


# speedup vs baseline: 1.5550x; 1.4326x over previous
import functools

import jax
import jax.numpy as jnp
from jax import lax
from jax.experimental import pallas as pl
from jax.experimental.pallas import tpu as pltpu, tpu_sc as plsc

NC = 2
NS = 16
NW = NC * NS
GRP = 128


def _ceil_to(x, m):
    return (x + m - 1) // m * m


def _make_sc_agg(n_nodes_pad, n_groups):
    rows_per_tile = n_nodes_pad // NS

    mesh = plsc.VectorSubcoreMesh(
        core_axis_name="c", subcore_axis_name="s", num_cores=NC, num_subcores=NS
    )
    out_type = [jax.ShapeDtypeStruct((n_nodes_pad, 128), jnp.float32),
                jax.ShapeDtypeStruct((n_nodes_pad, 128), jnp.float32)]
    scratch = [
        pltpu.VMEM((n_groups, GRP), jnp.int32),
        pltpu.VMEM((n_groups, GRP), jnp.int32),
        pltpu.VMEM((GRP, 128), jnp.float32),
        pltpu.VMEM_SHARED((n_nodes_pad, 128), jnp.float32),
        pltpu.SemaphoreType.DMA,
    ]

    def body(table, src3d, dst3d, zrows_hbm, agg_out0, agg_out1,
             srci, dsti, rows, acc, sem):
        c = lax.axis_index("c")
        s = lax.axis_index("s")
        wid = c * NS + s

        base_r = s * rows_per_tile
        pltpu.sync_copy(zrows_hbm, acc.at[pl.ds(base_r, rows_per_tile)])
        pltpu.sync_copy(src3d.at[wid], srci)
        pltpu.sync_copy(dst3d.at[wid], dsti)
        plsc.subcore_barrier()

        def step(j, carry):
            pltpu.async_copy(table.at[srci.at[j]], rows, sem).wait()
            pltpu.sync_copy(rows, acc.at[dsti.at[j]], add=True)
            return carry

        lax.fori_loop(0, n_groups, step, 0)
        plsc.subcore_barrier()
    return pl.kernel(body, out_type=out_type, mesh=mesh, scratch_types=scratch)


def _make_sc_pair_gather(n_pairs_pad, n_groups):
    mesh = plsc.VectorSubcoreMesh(
        core_axis_name="c", subcore_axis_name="s", num_cores=NC, num_subcores=NS
    )
    out_type = [
        jax.ShapeDtypeStruct((n_pairs_pad, 128), jnp.float32),
        jax.ShapeDtypeStruct((n_pairs_pad, 128), jnp.float32),
    ]
    scratch = [
        pltpu.VMEM((n_groups, GRP), jnp.int32),
        pltpu.VMEM((n_groups, GRP), jnp.int32),
        pltpu.VMEM((GRP, 128), jnp.float32),
        pltpu.VMEM((GRP, 128), jnp.float32),
        pltpu.SemaphoreType.DMA,
        pltpu.SemaphoreType.DMA,
    ]
    per_worker = n_groups * GRP

    def body(u, v, p0, p1, gu, gv, p0i, p1i, rows0a, rows0b, sem0, sem1):
        c = lax.axis_index("c")
        s = lax.axis_index("s")
        wid = c * NS + s
        pltpu.sync_copy(p0.at[wid], p0i)
        pltpu.sync_copy(p1.at[wid], p1i)

        def step(j, carry):
            out_r = wid * per_worker + j * GRP
            cp0 = pltpu.async_copy(u.at[p0i.at[j]], rows0a, sem0)
            cp1 = pltpu.async_copy(v.at[p1i.at[j]], rows0b, sem1)
            cp0.wait()
            pltpu.sync_copy(rows0a, gu.at[pl.ds(out_r, GRP)])
            cp1.wait()
            pltpu.sync_copy(rows0b, gv.at[pl.ds(out_r, GRP)])
            return carry

        lax.fori_loop(0, n_groups, step, 0)

    return pl.kernel(body, out_type=out_type, mesh=mesh, scratch_types=scratch)


def _deg_body(dst_ref, H):
    i = pl.program_id(0)

    @pl.when(i == 0)
    def _():
        H[...] = jnp.zeros_like(H)

    d = dst_ref[...]
    ids = lax.broadcasted_iota(jnp.int32, (d.shape[0], 128), 1)
    oh_hi = ((d // 128) == ids).astype(jnp.bfloat16)
    oh_lo = ((d % 128) == ids).astype(jnp.bfloat16)
    H[...] += lax.dot_general(oh_hi, oh_lo, (((0,), (0,)), ((), ())),
                              preferred_element_type=jnp.float32)


def _tc_degree(dst_col, eb=8000):
    e = dst_col.shape[0]
    return pl.pallas_call(
        _deg_body,
        grid=(e // eb,),
        in_specs=[pl.BlockSpec((eb, 1), lambda i: (i, 0))],
        out_specs=pl.BlockSpec((128, 128), lambda i: (0, 0)),
        out_shape=jax.ShapeDtypeStruct((128, 128), jnp.float32),
    )(dst_col)


def _layer1_body(a0, a1, dcol, x, wl, wr, b, o):
    deg = jnp.maximum(dcol[...], 1.0)
    mean = (a0[...] + a1[...]) / deg
    acc = jnp.dot(mean, wl[...], preferred_element_type=jnp.float32)
    acc += jnp.dot(x[...], wr[...], preferred_element_type=jnp.float32)
    o[...] = jnp.maximum(acc + b[...], 0.0)


def _layer2_body(a0, a1, dcol, x, wl, wr, b, wsa, wsb, u, v):
    deg = jnp.maximum(dcol[...], 1.0)
    mean = (a0[...] + a1[...]) / deg
    acc = jnp.dot(mean, wl[...], preferred_element_type=jnp.float32)
    acc += jnp.dot(x[...], wr[...], preferred_element_type=jnp.float32)
    h2 = jnp.maximum(acc + b[...], 0.0)
    u[...] = jnp.dot(h2, wsa[...], preferred_element_type=jnp.float32)
    v[...] = jnp.dot(h2, wsb[...], preferred_element_type=jnp.float32)


def _score_body(gu, gv, bs1, ws2, bs2, o):
    z = jnp.maximum(gu[...] + gv[...] + bs1[...], 0.0)
    s = jnp.dot(z, ws2[...], preferred_element_type=jnp.float32) + bs2[...]
    o[...] = 1.0 / (1.0 + jnp.exp(-s))


def _row_spec(rb, cols):
    return pl.BlockSpec((rb, cols), lambda i: (i, 0))


def _full_spec(shape):
    return pl.BlockSpec(shape, lambda i: (0,) * len(shape))


def _tc_layer1(a0, a1, dcol, x, wl, wr, b, rb=1000):
    n = x.shape[0]
    return pl.pallas_call(
        _layer1_body,
        grid=(n // rb,),
        in_specs=[_row_spec(rb, 128), _row_spec(rb, 128), _row_spec(rb, 1),
                  _row_spec(rb, 128), _full_spec((128, 128)),
                  _full_spec((128, 128)), _full_spec((1, 128))],
        out_specs=_row_spec(rb, 128),
        out_shape=jax.ShapeDtypeStruct((n, 128), jnp.float32),
    )(a0, a1, dcol, x, wl, wr, b)


def _tc_layer2(a0, a1, dcol, x, wl, wr, b, wsa, wsb, rb=1000):
    n = x.shape[0]
    return pl.pallas_call(
        _layer2_body,
        grid=(n // rb,),
        in_specs=[_row_spec(rb, 128), _row_spec(rb, 128), _row_spec(rb, 1),
                  _row_spec(rb, 128), _full_spec((128, 128)),
                  _full_spec((128, 128)), _full_spec((1, 128)),
                  _full_spec((128, 128)), _full_spec((128, 128))],
        out_specs=[_row_spec(rb, 128), _row_spec(rb, 128)],
        out_shape=[jax.ShapeDtypeStruct((n, 128), jnp.float32),
                   jax.ShapeDtypeStruct((n, 128), jnp.float32)],
    )(a0, a1, dcol, x, wl, wr, b, wsa, wsb)


def _tc_score(gu, gv, n, bs1, ws2, bs2, rb=1000):
    return pl.pallas_call(
        _score_body,
        grid=(n // rb,),
        in_specs=[_row_spec(rb, 128), _row_spec(rb, 128), _full_spec((1, 128)),
                  _full_spec((128, 1)), _full_spec((1, 1))],
        out_specs=_row_spec(rb, 1),
        out_shape=jax.ShapeDtypeStruct((n, 1), jnp.float32),
    )(gu, gv, bs1, ws2, bs2)


def kernel(x, edge_index, pairs, Wl1, Wr1, b1, Wl2, Wr2, b2, Ws1, bs1, Ws2, bs2):
    n_nodes = x.shape[0]
    n_edges = edge_index.shape[1]
    n_pairs = pairs.shape[0]

    e_pad = _ceil_to(n_edges, NW * GRP)
    e_groups = e_pad // (NW * GRP)
    node_pad = _ceil_to(n_nodes + 1, NS * 8)
    p_pad = _ceil_to(n_pairs, NW * GRP)
    p_groups = p_pad // (NW * GRP)

    src = jnp.pad(edge_index[0].astype(jnp.int32), (0, e_pad - n_edges),
                  constant_values=0).reshape(NW, e_groups, GRP)
    dst = jnp.pad(edge_index[1].astype(jnp.int32), (0, e_pad - n_edges),
                  constant_values=n_nodes).reshape(NW, e_groups, GRP)
    p0 = jnp.pad(pairs[:, 0].astype(jnp.int32), (0, p_pad - n_pairs),
                 constant_values=0).reshape(NW, p_groups, GRP)
    p1 = jnp.pad(pairs[:, 1].astype(jnp.int32), (0, p_pad - n_pairs),
                 constant_values=0).reshape(NW, p_groups, GRP)

    rows_per_tile = node_pad // NS
    zrows_hbm = jnp.zeros((rows_per_tile, 128), jnp.float32)

    sc_agg = _make_sc_agg(node_pad, e_groups)
    sc_pairs = _make_sc_pair_gather(p_pad, p_groups)

    a0, a1 = sc_agg(x, src, dst, zrows_hbm)
    H = _tc_degree(edge_index[1].astype(jnp.int32).reshape(n_edges, 1))
    dcol = H.reshape(-1)[:n_nodes, None]

    b1r = b1.reshape(1, 128)
    h = _tc_layer1(a0, a1, dcol, x, Wl1, Wr1, b1r)

    a20, a21 = sc_agg(h, src, dst, zrows_hbm)
    u, v = _tc_layer2(a20, a21, dcol, h, Wl2, Wr2, b2.reshape(1, 128),
                      Ws1[:128], Ws1[128:])

    gu, gv = sc_pairs(u, v, p0, p1)
    s = _tc_score(gu, gv, n_pairs, bs1.reshape(1, 128), Ws2, bs2.reshape(1, 1))
    return s.squeeze(-1)
